# baseline (speedup 1.0000x reference)
import jax
import jax.numpy as jnp
from jax.experimental import pallas as pl
from jax.experimental.pallas import tpu as pltpu

CH = 10000
NBUF = 3


def _body(x_hbm, m_ref, w_ref, b_ref, o_hbm, xbuf, obuf, rsem, wsem):
    i = pl.program_id(0)
    n = pl.num_programs(0)
    s = jax.lax.rem(i, NBUF)

    def rd(chunk, sl):
        return pltpu.make_async_copy(
            x_hbm.at[pl.ds(chunk * CH, CH), :], xbuf.at[sl], rsem.at[sl]
        )

    def wr(chunk, sl):
        return pltpu.make_async_copy(
            obuf.at[sl], o_hbm.at[pl.ds(chunk * CH, CH), :], wsem.at[sl]
        )

    @pl.when(i == 0)
    def _():
        rd(0, 0).start()
        rd(1, 1).start()

    @pl.when(i + 2 < n)
    def _():
        rd(i + 2, jax.lax.rem(i + 2, NBUF)).start()

    rd(i, s).wait()

    @pl.when(i >= NBUF)
    def _():
        wr(i - NBUF, s).wait()

    x = xbuf[s]
    y = jnp.dot(x, w_ref[...], preferred_element_type=jnp.float32) + b_ref[...]
    obuf[s] = jnp.where(m_ref[0, 0], y, x)
    wr(i, s).start()

    @pl.when(i == n - 1)
    def _():
        for j in range(NBUF):
            wr(n - NBUF + j, jax.lax.rem(n - NBUF + j, NBUF)).wait()


def kernel(in_tensor, mask, W, b):
    M, D = in_tensor.shape
    n = M // CH
    m4 = mask.reshape(n, 1, CH, 1)
    b2 = b.reshape(1, D)
    return pl.pallas_call(
        _body,
        grid=(n,),
        in_specs=[
            pl.BlockSpec(memory_space=pl.ANY),
            pl.BlockSpec((1, 1, CH, 1), lambda i: (i, 0, 0, 0)),
            pl.BlockSpec(memory_space=pltpu.VMEM),
            pl.BlockSpec(memory_space=pltpu.VMEM),
        ],
        out_specs=pl.BlockSpec(memory_space=pl.ANY),
        out_shape=jax.ShapeDtypeStruct((M, D), jnp.float32),
        scratch_shapes=[
            pltpu.VMEM((NBUF, CH, D), jnp.float32),
            pltpu.VMEM((NBUF, CH, D), jnp.float32),
            pltpu.SemaphoreType.DMA((NBUF,)),
            pltpu.SemaphoreType.DMA((NBUF,)),
        ],
        compiler_params=pltpu.CompilerParams(
            dimension_semantics=("arbitrary",),
        ),
    )(in_tensor, m4, W, b2)


# speedup vs baseline: 1.1922x; 1.1922x over previous
import jax
import jax.numpy as jnp
from jax.experimental import pallas as pl
from jax.experimental.pallas import tpu as pltpu


def _body(x_ref, m_ref, w_ref, b_ref, o_ref):
    x = x_ref[...]
    y = jnp.dot(x, w_ref[...], preferred_element_type=jnp.float32) + b_ref[...]
    o_ref[...] = jnp.where(m_ref[...], y, x)


def kernel(in_tensor, mask, W, b):
    M, D = in_tensor.shape
    BM = 8000
    mask2 = mask.reshape(M, 1)
    b2 = b.reshape(1, D)
    return pl.pallas_call(
        _body,
        grid=(M // BM,),
        in_specs=[
            pl.BlockSpec((BM, D), lambda i: (i, 0)),
            pl.BlockSpec((BM, 1), lambda i: (i, 0)),
            pl.BlockSpec((D, D), lambda i: (0, 0)),
            pl.BlockSpec((1, D), lambda i: (0, 0)),
        ],
        out_specs=pl.BlockSpec((BM, D), lambda i: (i, 0)),
        out_shape=jax.ShapeDtypeStruct((M, D), jnp.float32),
        compiler_params=pltpu.CompilerParams(
            dimension_semantics=("parallel",),
        ),
    )(in_tensor, mask2, W, b2)
